# baseline (speedup 1.0000x reference)
import dataclasses
import functools

import jax
import jax.numpy as jnp
from jax.experimental import pallas as pl
from jax.experimental.pallas import tpu as pltpu
from jax.experimental.pallas import tpu_sc as plsc

CP = 64
NCHAN = 96
NPIX = 512 * 512
CHUNK = 16384
LANES = 16


def _sc_call(cmc2, par2, img2):
    nrows = img2.shape[0]
    mesh = plsc.VectorSubcoreMesh(core_axis_name="c", subcore_axis_name="s")
    cp_params = pltpu.CompilerParams()
    if "needs_layout_passes" in pltpu.CompilerParams.__dataclass_fields__:
        cp_params = dataclasses.replace(cp_params, needs_layout_passes=False)

    @functools.partial(
        pl.kernel,
        out_type=jax.ShapeDtypeStruct((nrows, NPIX), jnp.float32),
        mesh=mesh,
        scratch_types=[pltpu.VMEM((2 * CP,), jnp.float32)],
        compiler_params=cp_params,
    )
    def run(cmc_hbm, par_hbm, img_hbm, out_hbm, ytab_ref):
        def body(cmc_v, par_v, img_v, out_v):
            lane = jax.lax.iota(jnp.int32, LANES)
            for t in range(CP // LANES):
                sl = pl.ds(t * LANES, LANES)
                ytab_ref[sl] = cmc_v[0, sl] + par_v[0, sl] * 0.04
            for t in range(CP // LANES):
                base = t * LANES
                nxt = jnp.minimum(lane + (base + 1), CP - 1)
                ynext = plsc.load_gather(ytab_ref, [nxt])
                ytab_ref[pl.ds(CP + base, LANES)] = (
                    ynext - ytab_ref[pl.ds(base, LANES)])
            for t in range(CP // LANES):
                sl = pl.ds(t * LANES, LANES)
                jf = (lane + t * LANES).astype(jnp.float32)
                ytab_ref[sl] = (
                    ytab_ref[sl] - jf * ytab_ref[pl.ds(CP + t * LANES, LANES)])

            @plsc.parallel_loop(0, CHUNK, step=LANES, unroll=6)
            def _(c0):
                sl = pl.ds(c0, LANES)
                x = img_v[0, sl]
                v = x * 63.0
                i = v.astype(jnp.int32)
                cc = plsc.load_gather(ytab_ref, [i])
                d = plsc.load_gather(ytab_ref, [i | CP])
                out_v[0, sl] = cc + v * d

        pltpu.emit_pipeline(
            body,
            grid=(nrows, NPIX // CHUNK),
            in_specs=[
                pl.BlockSpec((1, CP), lambda i, j: (i, 0)),
                pl.BlockSpec((1, CP), lambda i, j: (i, 0)),
                pl.BlockSpec((1, CHUNK), lambda i, j: (i, j)),
            ],
            out_specs=[pl.BlockSpec((1, CHUNK), lambda i, j: (i, j))],
            core_axis_name=("c", "s"),
            dimension_semantics=(pltpu.PARALLEL, pltpu.PARALLEL),
        )(cmc_hbm, par_hbm, img_hbm, out_hbm)

    return run(cmc2, par2, img2)


TC_BS = 2048


def _tc_body(cmc_v, par_v, img_v, out_v):
    tab = cmc_v[0] + par_v[0] * 0.04
    tab2 = jnp.concatenate([tab, tab], axis=-1)
    t = jnp.broadcast_to(tab2, (TC_BS, 128))
    x = img_v[0]
    v = x * 63.0
    i = jnp.minimum(v.astype(jnp.int32), 62)
    coeff = v - i.astype(jnp.float32)
    y0 = jnp.take_along_axis(t, i, axis=-1)
    y1 = jnp.take_along_axis(t, i + 1, axis=-1)
    out_v[0] = (1.0 - coeff) * y0 + coeff * y1


def _tc_call(cmc2, par2, img2):
    nrows = img2.shape[0]
    img3 = img2.reshape(nrows, NPIX // 128, 128)
    cmc3 = cmc2.reshape(nrows, 1, CP)
    par3 = par2.reshape(nrows, 1, CP)
    out = pl.pallas_call(
        _tc_body,
        grid=(nrows, (NPIX // 128) // TC_BS),
        in_specs=[
            pl.BlockSpec((1, 1, CP), lambda i, j: (i, 0, 0)),
            pl.BlockSpec((1, 1, CP), lambda i, j: (i, 0, 0)),
            pl.BlockSpec((1, TC_BS, 128), lambda i, j: (i, j, 0)),
        ],
        out_specs=pl.BlockSpec((1, TC_BS, 128), lambda i, j: (i, j, 0)),
        out_shape=jax.ShapeDtypeStruct((nrows, NPIX // 128, 128), jnp.float32),
    )(cmc3, par3, img3)
    return out.reshape(nrows, NPIX)


def kernel(org_img, params, color_map_control):
    N, C, H, W = org_img.shape
    img2 = org_img.reshape(NCHAN, NPIX)
    cmc2 = color_map_control.reshape(NCHAN, CP)
    par2 = params.reshape(NCHAN, CP)
    out = _sc_call(cmc2, par2, img2)
    return out.reshape(N, C, H, W)


# speedup vs baseline: 1.0642x; 1.0642x over previous
import dataclasses
import functools

import jax
import jax.numpy as jnp
from jax.experimental import pallas as pl
from jax.experimental.pallas import tpu as pltpu
from jax.experimental.pallas import tpu_sc as plsc

CP = 64
NCHAN = 96
NPIX = 512 * 512
CHUNK = 16384
LANES = 16


def _sc_call(cmc2, par2, img2):
    nrows = img2.shape[0]
    mesh = plsc.VectorSubcoreMesh(core_axis_name="c", subcore_axis_name="s")
    cp_params = pltpu.CompilerParams()
    if "needs_layout_passes" in pltpu.CompilerParams.__dataclass_fields__:
        cp_params = dataclasses.replace(cp_params, needs_layout_passes=False)

    @functools.partial(
        pl.kernel,
        out_type=jax.ShapeDtypeStruct((nrows, NPIX), jnp.float32),
        mesh=mesh,
        scratch_types=[pltpu.VMEM((2 * CP,), jnp.float32),
                       pltpu.VMEM((CP,), jnp.int32)],
        compiler_params=cp_params,
    )
    def run(cmc_hbm, par_hbm, img_hbm, out_hbm, ytab_ref, ptab_ref):
        def body(cmc_v, par_v, img_v, out_v):
            lane = jax.lax.iota(jnp.int32, LANES)
            for t in range(CP // LANES):
                sl = pl.ds(t * LANES, LANES)
                ytab_ref[sl] = cmc_v[0, sl] + par_v[0, sl] * 0.04
            for t in range(CP // LANES):
                base = t * LANES
                nxt = jnp.minimum(lane + (base + 1), CP - 1)
                ynext = plsc.load_gather(ytab_ref, [nxt])
                ytab_ref[pl.ds(CP + base, LANES)] = (
                    ynext - ytab_ref[pl.ds(base, LANES)])
            for t in range(CP // LANES):
                sl = pl.ds(t * LANES, LANES)
                yb = plsc.bitcast(ytab_ref[sl], jnp.int32)
                db = plsc.bitcast(ytab_ref[pl.ds(CP + t * LANES, LANES)],
                                  jnp.int32)
                rnd = jnp.int32(0x8000)
                ptab_ref[sl] = (
                    jax.lax.shift_right_logical(yb + rnd, 16)
                    | ((db + rnd) & jnp.int32(-65536)))

            @plsc.parallel_loop(0, CHUNK, step=LANES, unroll=6)
            def _(c0):
                sl = pl.ds(c0, LANES)
                x = img_v[0, sl]
                v = x * 63.0
                i = v.astype(jnp.int32)
                coeff = v - i.astype(jnp.float32)
                g = plsc.load_gather(ptab_ref, [i])
                y0 = plsc.bitcast(jax.lax.shift_left(g, 16), jnp.float32)
                d = plsc.bitcast(g, jnp.float32)
                out_v[0, sl] = y0 + coeff * d

        pltpu.emit_pipeline(
            body,
            grid=(nrows, NPIX // CHUNK),
            in_specs=[
                pl.BlockSpec((1, CP), lambda i, j: (i, 0)),
                pl.BlockSpec((1, CP), lambda i, j: (i, 0)),
                pl.BlockSpec((1, CHUNK), lambda i, j: (i, j)),
            ],
            out_specs=[pl.BlockSpec((1, CHUNK), lambda i, j: (i, j))],
            core_axis_name=("c", "s"),
            dimension_semantics=(pltpu.PARALLEL, pltpu.PARALLEL),
        )(cmc_hbm, par_hbm, img_hbm, out_hbm)

    return run(cmc2, par2, img2)


TC_BS = 2048


def _tc_body(cmc_v, par_v, img_v, out_v):
    tab = cmc_v[0] + par_v[0] * 0.04
    tab2 = jnp.concatenate([tab, tab], axis=-1)
    t = jnp.broadcast_to(tab2, (TC_BS, 128))
    x = img_v[0]
    v = x * 63.0
    i = jnp.minimum(v.astype(jnp.int32), 62)
    coeff = v - i.astype(jnp.float32)
    y0 = jnp.take_along_axis(t, i, axis=-1)
    y1 = jnp.take_along_axis(t, i + 1, axis=-1)
    out_v[0] = (1.0 - coeff) * y0 + coeff * y1


def _tc_call(cmc2, par2, img2):
    nrows = img2.shape[0]
    img3 = img2.reshape(nrows, NPIX // 128, 128)
    cmc3 = cmc2.reshape(nrows, 1, CP)
    par3 = par2.reshape(nrows, 1, CP)
    out = pl.pallas_call(
        _tc_body,
        grid=(nrows, (NPIX // 128) // TC_BS),
        in_specs=[
            pl.BlockSpec((1, 1, CP), lambda i, j: (i, 0, 0)),
            pl.BlockSpec((1, 1, CP), lambda i, j: (i, 0, 0)),
            pl.BlockSpec((1, TC_BS, 128), lambda i, j: (i, j, 0)),
        ],
        out_specs=pl.BlockSpec((1, TC_BS, 128), lambda i, j: (i, j, 0)),
        out_shape=jax.ShapeDtypeStruct((nrows, NPIX // 128, 128), jnp.float32),
    )(cmc3, par3, img3)
    return out.reshape(nrows, NPIX)


def kernel(org_img, params, color_map_control):
    N, C, H, W = org_img.shape
    img2 = org_img.reshape(NCHAN, NPIX)
    cmc2 = color_map_control.reshape(NCHAN, CP)
    par2 = params.reshape(NCHAN, CP)
    out = _sc_call(cmc2, par2, img2)
    return out.reshape(N, C, H, W)


# speedup vs baseline: 1.1457x; 1.0766x over previous
import dataclasses
import functools

import jax
import jax.numpy as jnp
from jax.experimental import pallas as pl
from jax.experimental.pallas import tpu as pltpu
from jax.experimental.pallas import tpu_sc as plsc

CP = 64
NCHAN = 96
NPIX = 512 * 512
CHUNK = 16384
LANES = 16


def _sc_call(cmc2, par2, img2):
    nrows = img2.shape[0]
    mesh = plsc.VectorSubcoreMesh(core_axis_name="c", subcore_axis_name="s")
    cp_params = pltpu.CompilerParams()
    if "needs_layout_passes" in pltpu.CompilerParams.__dataclass_fields__:
        cp_params = dataclasses.replace(cp_params, needs_layout_passes=False)

    @functools.partial(
        pl.kernel,
        out_type=jax.ShapeDtypeStruct((nrows, NPIX), jnp.float32),
        mesh=mesh,
        scratch_types=[pltpu.VMEM((2 * CP,), jnp.float32),
                       pltpu.VMEM((CP,), jnp.int32)],
        compiler_params=cp_params,
    )
    def run(cmc_hbm, par_hbm, img_hbm, out_hbm, ytab_ref, ptab_ref):
        def body(cmc_v, par_v, img_v, out_v):
            lane = jax.lax.iota(jnp.int32, LANES)
            for t in range(CP // LANES):
                sl = pl.ds(t * LANES, LANES)
                ytab_ref[sl] = cmc_v[0, sl] + par_v[0, sl] * 0.04
            for t in range(CP // LANES):
                base = t * LANES
                nxt = jnp.minimum(lane + (base + 1), CP - 1)
                ynext = plsc.load_gather(ytab_ref, [nxt])
                ytab_ref[pl.ds(CP + base, LANES)] = (
                    ynext - ytab_ref[pl.ds(base, LANES)])
            for t in range(CP // LANES):
                sl = pl.ds(t * LANES, LANES)
                yb = plsc.bitcast(ytab_ref[sl], jnp.int32)
                db = plsc.bitcast(ytab_ref[pl.ds(CP + t * LANES, LANES)],
                                  jnp.int32)
                rnd = jnp.int32(0x8000)
                ptab_ref[sl] = (
                    jax.lax.shift_right_logical(yb + rnd, 16)
                    | ((db + rnd) & jnp.int32(-65536)))

            @plsc.parallel_loop(0, CHUNK, step=LANES, unroll=6)
            def _(c0):
                sl = pl.ds(c0, LANES)
                x = img_v[0, sl]
                v = x * 63.0
                i = v.astype(jnp.int32)
                coeff = v - i.astype(jnp.float32)
                g = plsc.load_gather(ptab_ref, [i])
                y0 = plsc.bitcast(jax.lax.shift_left(g, 16), jnp.float32)
                d = plsc.bitcast(g, jnp.float32)
                out_v[0, sl] = y0 + coeff * d

        pltpu.emit_pipeline(
            body,
            grid=(nrows, NPIX // CHUNK),
            in_specs=[
                pl.BlockSpec((1, CP), lambda i, j: (i, 0)),
                pl.BlockSpec((1, CP), lambda i, j: (i, 0)),
                pl.BlockSpec((1, CHUNK), lambda i, j: (i, j)),
            ],
            out_specs=[pl.BlockSpec((1, CHUNK), lambda i, j: (i, j))],
            core_axis_name=("c", "s"),
            dimension_semantics=(pltpu.PARALLEL, pltpu.PARALLEL),
        )(cmc_hbm, par_hbm, img_hbm, out_hbm)

    return run(cmc2, par2, img2)


ROWS_PER_W = 3
NCHUNK = NPIX // CHUNK


def _sc_call_manual(cmc2, par2, img2):
    mesh = plsc.VectorSubcoreMesh(core_axis_name="c", subcore_axis_name="s")
    cp_params = pltpu.CompilerParams()
    if "needs_layout_passes" in pltpu.CompilerParams.__dataclass_fields__:
        cp_params = dataclasses.replace(cp_params, needs_layout_passes=False)
    nk = ROWS_PER_W * NCHUNK

    @functools.partial(
        pl.kernel,
        out_type=jax.ShapeDtypeStruct((NCHAN * NPIX,), jnp.float32),
        mesh=mesh,
        scratch_types=[
            pltpu.VMEM((CHUNK,), jnp.float32),
            pltpu.VMEM((CHUNK,), jnp.float32),
            pltpu.VMEM((CHUNK,), jnp.float32),
            pltpu.VMEM((CHUNK,), jnp.float32),
            pltpu.VMEM((ROWS_PER_W * CP,), jnp.float32),
            pltpu.VMEM((ROWS_PER_W * CP,), jnp.float32),
            pltpu.VMEM((2 * CP,), jnp.float32),
            pltpu.VMEM((ROWS_PER_W * CP,), jnp.int32),
            pltpu.SemaphoreType.DMA,
            pltpu.SemaphoreType.DMA,
            pltpu.SemaphoreType.DMA,
            pltpu.SemaphoreType.DMA,
        ],
        compiler_params=cp_params,
    )
    def run(cmc_hbm, par_hbm, img_hbm, out_hbm,
            in0, in1, ou0, ou1, cmcb, parb, ytab, ptab,
            si0, si1, so0, so1):
        inb, oub = (in0, in1), (ou0, ou1)
        sin, sout = (si0, si1), (so0, so1)
        wid = jax.lax.axis_index("s") * 2 + jax.lax.axis_index("c")
        row0 = wid * ROWS_PER_W
        pltpu.sync_copy(cmc_hbm.at[pl.ds(row0 * CP, ROWS_PER_W * CP)], cmcb)
        pltpu.sync_copy(par_hbm.at[pl.ds(row0 * CP, ROWS_PER_W * CP)], parb)

        lane = jax.lax.iota(jnp.int32, LANES)
        for r in range(ROWS_PER_W):
            for t in range(CP // LANES):
                sl = pl.ds(t * LANES, LANES)
                rsl = pl.ds(r * CP + t * LANES, LANES)
                ytab[sl] = cmcb[rsl] + parb[rsl] * 0.04
            for t in range(CP // LANES):
                base = t * LANES
                nxt = jnp.minimum(lane + (base + 1), CP - 1)
                ynext = plsc.load_gather(ytab, [nxt])
                ytab[pl.ds(CP + base, LANES)] = (
                    ynext - ytab[pl.ds(base, LANES)])
            rnd = jnp.int32(0x8000)
            for t in range(CP // LANES):
                sl = pl.ds(t * LANES, LANES)
                yb = plsc.bitcast(ytab[sl], jnp.int32)
                db = plsc.bitcast(ytab[pl.ds(CP + t * LANES, LANES)],
                                  jnp.int32)
                ptab[pl.ds(r * CP + t * LANES, LANES)] = (
                    jax.lax.shift_right_logical(yb + rnd, 16)
                    | ((db + rnd) & jnp.int32(-65536)))

        def in_copy(k, b):
            return pltpu.make_async_copy(
                img_hbm.at[pl.ds(row0 * NPIX + k * CHUNK, CHUNK)],
                inb[b], sin[b])

        def out_copy(k, b):
            return pltpu.make_async_copy(
                oub[b], out_hbm.at[pl.ds(row0 * NPIX + k * CHUNK, CHUNK)],
                sout[b])

        in_copy(0, 0).start()
        in_copy(1, 1).start()

        @pl.loop(0, nk, step=2)
        def _(kk):
            for b in range(2):
                k = kk + b
                bias = (k // NCHUNK) * CP
                in_copy(k, b).wait()

                @pl.when(kk > 0)
                def _():
                    out_copy(k - 2, b).wait()

                src, dst = inb[b], oub[b]

                @plsc.parallel_loop(0, CHUNK, step=LANES, unroll=6)
                def _(c0):
                    sl = pl.ds(c0, LANES)
                    x = src[sl]
                    v = x * 63.0
                    i0 = v.astype(jnp.int32)
                    coeff = v - i0.astype(jnp.float32)
                    g = plsc.load_gather(ptab, [i0 + bias])
                    y0 = plsc.bitcast(jax.lax.shift_left(g, 16), jnp.float32)
                    d = plsc.bitcast(g, jnp.float32)
                    dst[sl] = y0 + coeff * d

                @pl.when(kk + b + 2 < nk)
                def _():
                    in_copy(k + 2, b).start()

                out_copy(k, b).start()

        out_copy(nk - 2, 0).wait()
        out_copy(nk - 1, 1).wait()

    return run(cmc2, par2, img2)


TC_BS = 2048


def _tc_body(cmc_v, par_v, img_v, out_v):
    tab = cmc_v[0] + par_v[0] * 0.04
    tab2 = jnp.concatenate([tab, tab], axis=-1)
    t = jnp.broadcast_to(tab2, (TC_BS, 128))
    x = img_v[0]
    v = x * 63.0
    i = jnp.minimum(v.astype(jnp.int32), 62)
    coeff = v - i.astype(jnp.float32)
    y0 = jnp.take_along_axis(t, i, axis=-1)
    y1 = jnp.take_along_axis(t, i + 1, axis=-1)
    out_v[0] = (1.0 - coeff) * y0 + coeff * y1


def _tc_call(cmc2, par2, img2):
    nrows = img2.shape[0]
    img3 = img2.reshape(nrows, NPIX // 128, 128)
    cmc3 = cmc2.reshape(nrows, 1, CP)
    par3 = par2.reshape(nrows, 1, CP)
    out = pl.pallas_call(
        _tc_body,
        grid=(nrows, (NPIX // 128) // TC_BS),
        in_specs=[
            pl.BlockSpec((1, 1, CP), lambda i, j: (i, 0, 0)),
            pl.BlockSpec((1, 1, CP), lambda i, j: (i, 0, 0)),
            pl.BlockSpec((1, TC_BS, 128), lambda i, j: (i, j, 0)),
        ],
        out_specs=pl.BlockSpec((1, TC_BS, 128), lambda i, j: (i, j, 0)),
        out_shape=jax.ShapeDtypeStruct((nrows, NPIX // 128, 128), jnp.float32),
    )(cmc3, par3, img3)
    return out.reshape(nrows, NPIX)


def kernel(org_img, params, color_map_control):
    N, C, H, W = org_img.shape
    img2 = org_img.reshape(NCHAN * NPIX)
    cmc2 = color_map_control.reshape(NCHAN * CP)
    par2 = params.reshape(NCHAN * CP)
    out = _sc_call_manual(cmc2, par2, img2)
    return out.reshape(N, C, H, W)
